# Initial kernel scaffold; baseline (speedup 1.0000x reference)
import jax
import jax.numpy as jnp
from jax.experimental import pallas as pl


def kernel(x, edge_index, batch, W1, b1, W2, b2):
    raise NotImplementedError("write your pallas kernel here")


# speedup vs baseline: 163.7037x; 163.7037x over previous
import functools

import jax
import jax.numpy as jnp
from jax import lax
from jax.experimental import pallas as pl
from jax.experimental.pallas import tpu as pltpu
from jax.experimental.pallas import tpu_sc as plsc

N = 100000
E = 3200000
G = 256
EROWS = E // 128
NPAD = 100096
ZCH = NPAD // 16
NBW = 3136
F32 = jnp.float32
I32 = jnp.int32

_MESH = plsc.VectorSubcoreMesh(core_axis_name="c", subcore_axis_name="s")
_SC_PARAMS = pltpu.CompilerParams(needs_layout_passes=False,
                                 use_tc_tiling_on_sc=False)


def _iota16():
    return lax.iota(I32, 16)


def _dinv_body(degp_ref, dinv_ref):
    d0 = degp_ref[pl.ds(0, NPAD)]
    d1 = degp_ref[pl.ds(NPAD, NPAD)]
    dinv_ref[...] = lax.rsqrt(d0 + d1 + 1.0)


@functools.partial(
    pl.kernel,
    out_type=jax.ShapeDtypeStruct((2 * NPAD,), F32),
    mesh=_MESH,
    compiler_params=_SC_PARAMS,
    scratch_types=[
        pltpu.VMEM((16, 128), I32),
        pltpu.VMEM((16, 128), I32),
        pltpu.VMEM((128,), F32),
        pltpu.VMEM((ZCH,), F32),
        pltpu.VMEM_SHARED((NPAD,), F32),
        pltpu.SemaphoreType.DMA,
        pltpu.SemaphoreType.DMA,
    ],
)
def _deg_kernel(e2d, degp, idx0, idx1, ones, zbuf, deg_sh, sem0, sem1):
    c = lax.axis_index("c")
    s = lax.axis_index("s")

    def _z(t, _):
        zbuf[pl.ds(t * 16, 16)] = jnp.zeros((16,), F32)
        return 0

    lax.fori_loop(0, ZCH // 16, _z, 0)
    for k in range(8):
        ones[pl.ds(k * 16, 16)] = jnp.full((16,), 1.0, F32)
    pltpu.sync_copy(zbuf, deg_sh.at[pl.ds(s * ZCH, ZCH)])
    plsc.subcore_barrier()

    extra = 10 + c
    start_o = c * 1562 + s * 97 + jnp.minimum(s, extra)
    cnt_o = 97 + (s < extra).astype(I32)
    sets = [(idx0, sem0), (idx1, sem1)]

    def _j0(b):
        r0_o = jnp.minimum(start_o + b * 2, start_o + cnt_o - 2)
        return (start_o + b * 2 - r0_o) * 8, r0_o * 8

    def _stage_fire(b, st):
        idx, sem = st
        j0, r0 = _j0(b)
        pltpu.sync_copy(e2d.at[pl.ds(EROWS + r0, 16)], idx)

        def _row(j, _):
            pltpu.async_copy(ones, deg_sh.at[idx.at[j]], sem, add=True)
            return 0

        lax.fori_loop(j0, 16, _row, 0)

    def _drain(b, st):
        idx, sem = st
        j0, _ = _j0(b)

        def _row(j, _):
            pltpu.make_async_copy(ones, deg_sh.at[idx.at[j]], sem).wait()
            return 0

        lax.fori_loop(j0, 16, _row, 0)

    _stage_fire(jnp.int32(0), sets[0])
    _stage_fire(jnp.int32(1), sets[1])

    def _pair(bb, _):
        for p in range(2):
            b = bb * 2 + p
            _drain(b, sets[p])

            @pl.when(b + 2 < 49)
            def _():
                _stage_fire(b + 2, sets[p])
        return 0

    lax.fori_loop(0, 24, _pair, 0)
    _drain(jnp.int32(48), sets[0])
    plsc.subcore_barrier()
    pltpu.sync_copy(deg_sh.at[pl.ds(s * ZCH, ZCH)], zbuf)
    pltpu.sync_copy(zbuf, degp.at[pl.ds(c * NPAD + s * ZCH, ZCH)])


@functools.partial(
    pl.kernel,
    out_type=jax.ShapeDtypeStruct((N, 8), F32),
    mesh=_MESH,
    compiler_params=_SC_PARAMS,
    scratch_types=[
        pltpu.VMEM((NBW,), F32),
        pltpu.VMEM((NBW, 3), F32),
        pltpu.VMEM((NBW,), I32),
        pltpu.VMEM((NBW, 8), F32),
    ],
)
def _table_kernel(dinv_hbm, x, batch, tout, dv, xb, bb, tb):
    c = lax.axis_index("c")
    s = lax.axis_index("s")
    w = s * 2 + c
    base = jnp.minimum(w * NBW, N - NBW)
    i16 = _iota16()
    pltpu.sync_copy(dinv_hbm.at[pl.ds(base, NBW)], dv)
    pltpu.sync_copy(x.at[pl.ds(base, NBW)], xb)
    pltpu.sync_copy(batch.at[pl.ds(base, NBW)], bb)

    def _step(t, _):
        rows = i16 + t * 16
        dinv = dv[pl.ds(t * 16, 16)]
        bf = plsc.load_gather(bb, [rows]).astype(F32)
        for col in range(3):
            xc = plsc.load_gather(xb, [rows, jnp.full((16,), col, I32)])
            plsc.store_scatter(tb, [rows, jnp.full((16,), col, I32)],
                               xc * dinv)
        plsc.store_scatter(tb, [rows, jnp.full((16,), 3, I32)], dinv)
        plsc.store_scatter(tb, [rows, jnp.full((16,), 4, I32)], bf)
        return 0

    lax.fori_loop(0, NBW // 16, _step, 0)
    pltpu.sync_copy(tb, tout.at[pl.ds(base, NBW)])


@functools.partial(
    pl.kernel,
    out_type=jax.ShapeDtypeStruct((32, 64, 256), F32),
    mesh=_MESH,
    compiler_params=_SC_PARAMS,
    scratch_types=[
        pltpu.VMEM((16, 128), I32),
        pltpu.VMEM((16, 128), I32),
        pltpu.VMEM((16, 128), I32),
        pltpu.VMEM((16, 128), I32),
        pltpu.VMEM((2048, 8), F32),
        pltpu.VMEM((2048, 8), F32),
        pltpu.VMEM((2048, 8), F32),
        pltpu.VMEM((2048, 8), F32),
        pltpu.VMEM((64, 256), F32),
        pltpu.VMEM((NBW, 8), F32),
        pltpu.SemaphoreType.DMA,
        pltpu.SemaphoreType.DMA,
    ],
)
def _acc_kernel(tin, e2d, parts, sidx0, sidx1, didx0, didx1, srows0, srows1,
                drows0, drows1, acc, nb, sem0, sem1):
    c = lax.axis_index("c")
    s = lax.axis_index("s")
    w = s * 2 + c
    i16 = _iota16()
    c3 = jnp.full((16,), 3, I32)
    c4 = jnp.full((16,), 4, I32)
    rv = [i16 + (col * 16) for col in range(4)]
    onesf = jnp.full((16,), 1.0, F32)
    sets = [(sidx0, didx0, srows0, drows0, sem0),
            (sidx1, didx1, srows1, drows1, sem1)]

    def _za(t, _):
        r = t // 16
        k = t % 16
        acc[r, pl.ds(k * 16, 16)] = jnp.zeros((16,), F32)
        return 0

    lax.fori_loop(0, 64 * 16, _za, 0)

    start_o = w * 97 + jnp.minimum(w, 21)
    cnt_o = 97 + (w < 21).astype(I32)
    nblk = 49

    def _j0(b):
        r0_o = jnp.minimum(start_o + b * 2, start_o + cnt_o - 2)
        return (start_o + b * 2 - r0_o) * 8, r0_o * 8

    def _stage_fire(b, st):
        sidx, didx, srows, drows, sem = st
        j0, r0 = _j0(b)
        pltpu.sync_copy(e2d.at[pl.ds(r0, 16)], sidx)
        pltpu.sync_copy(e2d.at[pl.ds(EROWS + r0, 16)], didx)

        def _fire(j, _):
            pltpu.async_copy(tin.at[sidx.at[j]],
                             srows.at[pl.ds(j * 128, 128)], sem)
            pltpu.async_copy(tin.at[didx.at[j]],
                             drows.at[pl.ds(j * 128, 128)], sem)
            return 0

        lax.fori_loop(j0, 16, _fire, 0)

    def _drain_compute(b, st):
        sidx, didx, srows, drows, sem = st
        j0, _ = _j0(b)

        def _drain(j, _):
            pltpu.make_async_copy(tin.at[sidx.at[j]],
                                  srows.at[pl.ds(j * 128, 128)], sem).wait()
            pltpu.make_async_copy(tin.at[didx.at[j]],
                                  drows.at[pl.ds(j * 128, 128)], sem).wait()
            return 0

        lax.fori_loop(j0, 16, _drain, 0)

        def _vec(t, _):
            rows = i16 + t * 16
            dinv = plsc.load_gather(drows, [rows, c3])
            g = plsc.load_gather(drows, [rows, c4]).astype(I32)
            for col in range(3):
                yc = plsc.load_gather(srows,
                                      [rows, jnp.full((16,), col, I32)])
                plsc.addupdate_scatter(acc, [rv[col], g], yc * dinv)
            return 0

        lax.fori_loop(j0 * 8, 128, _vec, 0)

    _stage_fire(jnp.int32(0), sets[0])
    _stage_fire(jnp.int32(1), sets[1])

    def _pair(bb, _):
        for p in range(2):
            b = bb * 2 + p
            _drain_compute(b, sets[p])

            @pl.when(b + 2 < nblk)
            def _():
                _stage_fire(b + 2, sets[p])
        return 0

    lax.fori_loop(0, nblk // 2, _pair, 0)
    _drain_compute(jnp.int32(nblk - 1), sets[0])

    nbase = jnp.minimum(w * NBW, N - NBW)
    t0 = (w * NBW - nbase) // 16
    pltpu.sync_copy(tin.at[pl.ds(nbase, NBW)], nb)

    def _nstep(t, _):
        rows = i16 + t * 16
        dinv = plsc.load_gather(nb, [rows, c3])
        g = plsc.load_gather(nb, [rows, c4]).astype(I32)
        for col in range(3):
            yc = plsc.load_gather(nb, [rows, jnp.full((16,), col, I32)])
            plsc.addupdate_scatter(acc, [rv[col], g], yc * dinv)
        plsc.addupdate_scatter(acc, [rv[3], g], onesf)
        return 0

    lax.fori_loop(t0, NBW // 16, _nstep, 0)
    pltpu.sync_copy(acc, parts.at[w])


def _final_body(parts_ref, w1_ref, b1_ref, w2_ref, b2_ref, out_ref):
    p = parts_ref[...]
    sums = jnp.sum(p, axis=0)
    s4 = jnp.sum(sums.reshape(4, 16, 256), axis=1)
    a = jnp.dot(w1_ref[...], w2_ref[...],
                preferred_element_type=F32)
    cvec = jnp.dot(b1_ref[...].reshape(1, 128), w2_ref[...],
                   preferred_element_type=F32)
    out = b2_ref[...][None, :]
    for k in range(3):
        out = out + s4[k].reshape(256, 1) * a[k].reshape(1, 5)
    out = out + s4[3].reshape(256, 1) * cvec
    out_ref[...] = out


def kernel(x, edge_index, batch, W1, b1, W2, b2):
    e2d = edge_index.reshape(2 * EROWS, 128)
    degp = _deg_kernel(e2d)
    dinv = pl.pallas_call(
        _dinv_body,
        out_shape=jax.ShapeDtypeStruct((NPAD,), F32),
    )(degp)
    table = _table_kernel(dinv, x, batch)
    parts = _acc_kernel(table, e2d)
    out = pl.pallas_call(
        _final_body,
        out_shape=jax.ShapeDtypeStruct((G, 5), F32),
    )(parts, W1, b1, W2, b2)
    return out
